# baseline (speedup 1.0000x reference)
import functools

import jax
import jax.numpy as jnp
from jax import lax
from jax.experimental import pallas as pl
from jax.experimental.pallas import tpu as pltpu
from jax.experimental.pallas import tpu_sc as plsc

_N_NODES = 100000
_N_EDGES = 6400000
_N_GRAPHS = 128

_NC, _NS, _L = 2, 16, 16
_NW = _NC * _NS
_CHUNK = 1024
_NCHUNKS = _N_EDGES // _CHUNK
_CPW = -(-_NCHUNKS // _NW)


def _sc_body(tx_h, ty_h, tz_h, tb_h, src_hbm, dst_hbm, out_hbm,
             tx, ty, tz, tb, sidx, didx,
             sxb, syb, szb, sbb, dxb, dyb, dzb,
             bins_ev, bins_cnt, sem):
    c = lax.axis_index("c")
    s = lax.axis_index("s")
    w = s * _NC + c

    @pl.when(s == 0)
    def _stage():
        pltpu.sync_copy(tx_h, tx)
        pltpu.sync_copy(ty_h, ty)
        pltpu.sync_copy(tz_h, tz)
        pltpu.sync_copy(tb_h, tb)

    plsc.subcore_barrier()

    zeros = jnp.zeros((_L,), jnp.float32)

    def _zero(i, carry):
        bins_ev[pl.ds(i * _L, _L)] = zeros
        bins_cnt[pl.ds(i * _L, _L)] = zeros
        return carry

    lax.fori_loop(0, _N_GRAPHS, _zero, 0)

    lane = lax.iota(jnp.int32, _L)
    ones = jnp.ones((_L,), jnp.float32)

    def _chunk(i, carry):
        cid = w + i * _NW

        @pl.when(cid < _NCHUNKS)
        def _do():
            base = cid * _CHUNK
            pltpu.sync_copy(src_hbm.at[pl.ds(base, _CHUNK)], sidx)
            pltpu.sync_copy(dst_hbm.at[pl.ds(base, _CHUNK)], didx)
            copies = [
                pltpu.async_copy(tx.at[sidx], sxb, sem),
                pltpu.async_copy(ty.at[sidx], syb, sem),
                pltpu.async_copy(tz.at[sidx], szb, sem),
                pltpu.async_copy(tb.at[sidx], sbb, sem),
                pltpu.async_copy(tx.at[didx], dxb, sem),
                pltpu.async_copy(ty.at[didx], dyb, sem),
                pltpu.async_copy(tz.at[didx], dzb, sem),
            ]
            for cp in copies:
                cp.wait()

            def _vec(k, carry2):
                sl = pl.ds(k * _L, _L)
                dx = dxb[sl] - sxb[sl]
                dy = dyb[sl] - syb[sl]
                dz = dzb[sl] - szb[sl]
                ss = dx * dx + dy * dy + dz * dz
                sc = jnp.maximum(ss, 1e-30)
                yi = 0x5F3759DF - (lax.bitcast_convert_type(sc, jnp.int32)
                                   >> 1)
                y = lax.bitcast_convert_type(yi, jnp.float32)
                hs = 0.5 * sc
                y = y * (1.5 - hs * y * y)
                y = y * (1.5 - hs * y * y)
                y = y * (1.5 - hs * y * y)
                eu = ss * y
                em = eu - 1.0
                ev = em * em
                gi = sbb[sl] * _L + lane
                plsc.addupdate_scatter(bins_ev, [gi], ev)
                plsc.addupdate_scatter(bins_cnt, [gi], ones)
                return carry2

            lax.fori_loop(0, _CHUNK // _L, _vec, 0)

        return carry

    lax.fori_loop(0, _CPW, _chunk, 0)

    pltpu.sync_copy(bins_ev, out_hbm.at[w, 0])
    pltpu.sync_copy(bins_cnt, out_hbm.at[w, 1])


def _finish_body(ev_ref, cnt_ref, o_ref):
    ev = jnp.sum(ev_ref[...], axis=0)
    cnt = jnp.sum(cnt_ref[...], axis=0)
    evg = jnp.sum(ev, axis=1, keepdims=True)
    cg = jnp.sum(cnt, axis=1, keepdims=True)
    gv = jnp.where(cg > 0, evg / jnp.maximum(cg, 1.0), 0.0)
    o_ref[...] = (jnp.sum(gv) / jnp.float32(_N_GRAPHS)).reshape(1, 1)


@jax.jit
def _run(tx, ty, tz, tb, src1d, dst1d):
    mesh = plsc.VectorSubcoreMesh(core_axis_name="c", subcore_axis_name="s")
    partials = pl.kernel(
        _sc_body,
        out_type=jax.ShapeDtypeStruct((_NW, 2, _N_GRAPHS * _L), jnp.float32),
        mesh=mesh,
        compiler_params=pltpu.CompilerParams(needs_layout_passes=False),
        scratch_types=[
            pltpu.VMEM_SHARED((_N_NODES,), jnp.float32),
            pltpu.VMEM_SHARED((_N_NODES,), jnp.float32),
            pltpu.VMEM_SHARED((_N_NODES,), jnp.float32),
            pltpu.VMEM_SHARED((_N_NODES,), jnp.int32),
            pltpu.VMEM((_CHUNK,), jnp.int32),
            pltpu.VMEM((_CHUNK,), jnp.int32),
            pltpu.VMEM((_CHUNK,), jnp.float32),
            pltpu.VMEM((_CHUNK,), jnp.float32),
            pltpu.VMEM((_CHUNK,), jnp.float32),
            pltpu.VMEM((_CHUNK,), jnp.int32),
            pltpu.VMEM((_CHUNK,), jnp.float32),
            pltpu.VMEM((_CHUNK,), jnp.float32),
            pltpu.VMEM((_CHUNK,), jnp.float32),
            pltpu.VMEM((_N_GRAPHS * _L,), jnp.float32),
            pltpu.VMEM((_N_GRAPHS * _L,), jnp.float32),
            pltpu.SemaphoreType.DMA,
        ],
    )(tx, ty, tz, tb, src1d, dst1d)

    ev_part = partials[:, 0, :].reshape(_NW, _N_GRAPHS, _L)
    cnt_part = partials[:, 1, :].reshape(_NW, _N_GRAPHS, _L)
    res = pl.pallas_call(
        _finish_body,
        out_shape=jax.ShapeDtypeStruct((1, 1), jnp.float32),
    )(ev_part, cnt_part)
    return res[0, 0]


def kernel(node_pos, raw_edge_index, batch):
    ei = raw_edge_index.astype(jnp.int32)
    pos = node_pos.astype(jnp.float32)
    return _run(pos[:, 0], pos[:, 1], pos[:, 2], batch.astype(jnp.int32),
                ei[0], ei[1])


# speedup vs baseline: 2.1362x; 2.1362x over previous
import functools

import jax
import jax.numpy as jnp
from jax import lax
from jax.experimental import pallas as pl
from jax.experimental.pallas import tpu as pltpu
from jax.experimental.pallas import tpu_sc as plsc

_N_NODES = 100000
_N_EDGES = 6400000
_N_GRAPHS = 128

_NC, _NS, _L = 2, 16, 16
_NW = _NC * _NS
_CHUNK = 1024
_NCHUNKS = _N_EDGES // _CHUNK
_CPW = -(-_NCHUNKS // _NW)


def _sc_body(tx_h, ty_h, tz_h, tb_h, src_hbm, dst_hbm, out_hbm,
             tx, ty, tz, tb, sidx0, didx0, sidx1, didx1,
             cols0, cols1, bins_ev, bins_cnt,
             semA0, semA1, semB0, semB1):
    c = lax.axis_index("c")
    s = lax.axis_index("s")
    w = s * _NC + c

    @pl.when(s == 0)
    def _stage():
        pltpu.sync_copy(tx_h, tx)
        pltpu.sync_copy(ty_h, ty)
        pltpu.sync_copy(tz_h, tz)
        pltpu.sync_copy(tb_h, tb)

    plsc.subcore_barrier()

    zeros = jnp.zeros((_L,), jnp.float32)

    def _zero(i, carry):
        bins_ev[pl.ds(i * _L, _L)] = zeros
        bins_cnt[pl.ds(i * _L, _L)] = zeros
        return carry

    lax.fori_loop(0, _N_GRAPHS, _zero, 0)

    lane = lax.iota(jnp.int32, _L)
    ones = jnp.ones((_L,), jnp.float32)

    def _valid(n):
        return (w + n * _NW) < _NCHUNKS

    def _idx_copies(n, sidx_, didx_, semb):
        base = (w + n * _NW) * _CHUNK
        return [
            pltpu.make_async_copy(src_hbm.at[pl.ds(base, _CHUNK)], sidx_,
                                  semb),
            pltpu.make_async_copy(dst_hbm.at[pl.ds(base, _CHUNK)], didx_,
                                  semb),
        ]

    def _gather_copies(sidx_, didx_, cols, sema):
        return [
            pltpu.make_async_copy(tx.at[sidx_], cols.at[pl.ds(0, _CHUNK)],
                                  sema),
            pltpu.make_async_copy(ty.at[sidx_],
                                  cols.at[pl.ds(_CHUNK, _CHUNK)], sema),
            pltpu.make_async_copy(tz.at[sidx_],
                                  cols.at[pl.ds(2 * _CHUNK, _CHUNK)], sema),
            pltpu.make_async_copy(tb.at[sidx_],
                                  cols.at[pl.ds(3 * _CHUNK, _CHUNK)], sema),
            pltpu.make_async_copy(tx.at[didx_],
                                  cols.at[pl.ds(4 * _CHUNK, _CHUNK)], sema),
            pltpu.make_async_copy(ty.at[didx_],
                                  cols.at[pl.ds(5 * _CHUNK, _CHUNK)], sema),
            pltpu.make_async_copy(tz.at[didx_],
                                  cols.at[pl.ds(6 * _CHUNK, _CHUNK)], sema),
        ]

    def _start(copies):
        for cp in copies:
            cp.start()

    def _wait(copies):
        for cp in copies:
            cp.wait()

    def _compute(cols):
        def _vec(k, carry2):
            sl = pl.ds(k * _L, _L)
            dx = cols[pl.ds(4 * _CHUNK + k * _L, _L)] - cols[sl]
            dy = (cols[pl.ds(5 * _CHUNK + k * _L, _L)]
                  - cols[pl.ds(_CHUNK + k * _L, _L)])
            dz = (cols[pl.ds(6 * _CHUNK + k * _L, _L)]
                  - cols[pl.ds(2 * _CHUNK + k * _L, _L)])
            ss = dx * dx + dy * dy + dz * dz
            sc = jnp.maximum(ss, 1e-30)
            yi = 0x5F3759DF - (lax.bitcast_convert_type(sc, jnp.int32) >> 1)
            y = lax.bitcast_convert_type(yi, jnp.float32)
            hs = 0.5 * sc
            y = y * (1.5 - hs * y * y)
            y = y * (1.5 - hs * y * y)
            y = y * (1.5 - hs * y * y)
            eu = ss * y
            em = eu - 1.0
            ev = em * em
            gb = cols[pl.ds(3 * _CHUNK + k * _L, _L)]
            gi = gb.astype(jnp.int32) * _L + lane
            plsc.addupdate_scatter(bins_ev, [gi], ev)
            plsc.addupdate_scatter(bins_cnt, [gi], ones)
            return carry2

        lax.fori_loop(0, _CHUNK // _L, _vec, 0)

    _start(_idx_copies(0, sidx0, didx0, semB0))
    _wait(_idx_copies(0, sidx0, didx0, semB0))
    _start(_gather_copies(sidx0, didx0, cols0, semA0))
    _start(_idx_copies(1, sidx1, didx1, semB1))

    def _pipe(i, carry):
        n0 = 2 * i
        n1 = n0 + 1

        @pl.when(_valid(n1))
        def _s1():
            _wait(_idx_copies(n1, sidx1, didx1, semB1))
            _start(_gather_copies(sidx1, didx1, cols1, semA1))

        @pl.when(_valid(n0))
        def _s2():
            _wait(_gather_copies(sidx0, didx0, cols0, semA0))

        @pl.when(_valid(n0 + 2))
        def _s3():
            _start(_idx_copies(n0 + 2, sidx0, didx0, semB0))

        @pl.when(_valid(n0))
        def _s4():
            _compute(cols0)

        @pl.when(_valid(n0 + 2))
        def _s5():
            _wait(_idx_copies(n0 + 2, sidx0, didx0, semB0))
            _start(_gather_copies(sidx0, didx0, cols0, semA0))

        @pl.when(_valid(n1))
        def _s6():
            _wait(_gather_copies(sidx1, didx1, cols1, semA1))

        @pl.when(_valid(n1 + 2))
        def _s7():
            _start(_idx_copies(n1 + 2, sidx1, didx1, semB1))

        @pl.when(_valid(n1))
        def _s8():
            _compute(cols1)

        return carry

    lax.fori_loop(0, _CPW // 2, _pipe, 0)

    pltpu.sync_copy(bins_ev, out_hbm.at[w, 0])
    pltpu.sync_copy(bins_cnt, out_hbm.at[w, 1])


def _finish_body(ev_ref, cnt_ref, o_ref):
    ev = jnp.sum(ev_ref[...], axis=0)
    cnt = jnp.sum(cnt_ref[...], axis=0)
    evg = jnp.sum(ev, axis=1, keepdims=True)
    cg = jnp.sum(cnt, axis=1, keepdims=True)
    gv = jnp.where(cg > 0, evg / jnp.maximum(cg, 1.0), 0.0)
    o_ref[...] = (jnp.sum(gv) / jnp.float32(_N_GRAPHS)).reshape(1, 1)


@jax.jit
def _run(tx, ty, tz, tb, src1d, dst1d):
    mesh = plsc.VectorSubcoreMesh(core_axis_name="c", subcore_axis_name="s")
    partials = pl.kernel(
        _sc_body,
        out_type=jax.ShapeDtypeStruct((_NW, 2, _N_GRAPHS * _L), jnp.float32),
        mesh=mesh,
        compiler_params=pltpu.CompilerParams(needs_layout_passes=False),
        scratch_types=[
            pltpu.VMEM_SHARED((_N_NODES,), jnp.float32),
            pltpu.VMEM_SHARED((_N_NODES,), jnp.float32),
            pltpu.VMEM_SHARED((_N_NODES,), jnp.float32),
            pltpu.VMEM_SHARED((_N_NODES,), jnp.float32),
            pltpu.VMEM((_CHUNK,), jnp.int32),
            pltpu.VMEM((_CHUNK,), jnp.int32),
            pltpu.VMEM((_CHUNK,), jnp.int32),
            pltpu.VMEM((_CHUNK,), jnp.int32),
            pltpu.VMEM((7 * _CHUNK,), jnp.float32),
            pltpu.VMEM((7 * _CHUNK,), jnp.float32),
            pltpu.VMEM((_N_GRAPHS * _L,), jnp.float32),
            pltpu.VMEM((_N_GRAPHS * _L,), jnp.float32),
            pltpu.SemaphoreType.DMA,
            pltpu.SemaphoreType.DMA,
            pltpu.SemaphoreType.DMA,
            pltpu.SemaphoreType.DMA,
        ],
    )(tx, ty, tz, tb, src1d, dst1d)

    ev_part = partials[:, 0, :].reshape(_NW, _N_GRAPHS, _L)
    cnt_part = partials[:, 1, :].reshape(_NW, _N_GRAPHS, _L)
    res = pl.pallas_call(
        _finish_body,
        out_shape=jax.ShapeDtypeStruct((1, 1), jnp.float32),
    )(ev_part, cnt_part)
    return res[0, 0]


def kernel(node_pos, raw_edge_index, batch):
    ei = raw_edge_index.astype(jnp.int32)
    pos = node_pos.astype(jnp.float32)
    return _run(pos[:, 0], pos[:, 1], pos[:, 2], batch.astype(jnp.float32),
                ei[0], ei[1])


# speedup vs baseline: 2.7560x; 1.2901x over previous
import functools

import jax
import jax.numpy as jnp
from jax import lax
from jax.experimental import pallas as pl
from jax.experimental.pallas import tpu as pltpu
from jax.experimental.pallas import tpu_sc as plsc

_N_NODES = 100000
_N_EDGES = 6400000
_N_GRAPHS = 128

_NC, _NS, _L = 2, 16, 16
_NW = _NC * _NS
_CHUNK = 1024
_NCHUNKS = _N_EDGES // _CHUNK
_CPW = -(-_NCHUNKS // _NW)


def _sc_body(ta_h, tb_h, src_hbm, dst_hbm, out_hbm,
             ta, tb, sidx0, didx0, sidx1, didx1,
             cols0, cols1, bins_ev, bins_cnt,
             semA0, semA1, semB0, semB1):
    c = lax.axis_index("c")
    s = lax.axis_index("s")
    w = s * _NC + c

    @pl.when(s == 0)
    def _stage():
        pltpu.sync_copy(ta_h, ta)
        pltpu.sync_copy(tb_h, tb)

    plsc.subcore_barrier()

    zeros = jnp.zeros((_L,), jnp.float32)

    def _zero(i, carry):
        bins_ev[pl.ds(i * _L, _L)] = zeros
        bins_cnt[pl.ds(i * _L, _L)] = zeros
        return carry

    lax.fori_loop(0, _N_GRAPHS, _zero, 0)

    lane = lax.iota(jnp.int32, _L)
    ones = jnp.ones((_L,), jnp.float32)

    def _valid(n):
        return (w + n * _NW) < _NCHUNKS

    def _idx_copies(n, sidx_, didx_, semb):
        base = (w + n * _NW) * _CHUNK
        return [
            pltpu.make_async_copy(src_hbm.at[pl.ds(base, _CHUNK)], sidx_,
                                  semb),
            pltpu.make_async_copy(dst_hbm.at[pl.ds(base, _CHUNK)], didx_,
                                  semb),
        ]

    def _gather_copies(sidx_, didx_, cols, sema):
        return [
            pltpu.make_async_copy(ta.at[sidx_], cols.at[pl.ds(0, _CHUNK)],
                                  sema),
            pltpu.make_async_copy(tb.at[sidx_],
                                  cols.at[pl.ds(_CHUNK, _CHUNK)], sema),
            pltpu.make_async_copy(ta.at[didx_],
                                  cols.at[pl.ds(2 * _CHUNK, _CHUNK)], sema),
            pltpu.make_async_copy(tb.at[didx_],
                                  cols.at[pl.ds(3 * _CHUNK, _CHUNK)], sema),
        ]

    def _start(copies):
        for cp in copies:
            cp.start()

    def _wait(copies):
        for cp in copies:
            cp.wait()

    _HI = jnp.int32(-65536)
    _LO = jnp.int32(65535)

    def _bcf(v):
        return lax.bitcast_convert_type(v, jnp.float32)

    def _compute(cols):
        def _vec(k, carry2):
            vas = cols[pl.ds(k * _L, _L)]
            vbs = cols[pl.ds(_CHUNK + k * _L, _L)]
            vad = cols[pl.ds(2 * _CHUNK + k * _L, _L)]
            vbd = cols[pl.ds(3 * _CHUNK + k * _L, _L)]
            dx = _bcf(vad << 16) - _bcf(vas << 16)
            dy = _bcf(vad & _HI) - _bcf(vas & _HI)
            dz = _bcf(vbd & _HI) - _bcf(vbs & _HI)
            ss = dx * dx + dy * dy + dz * dz
            sc = jnp.maximum(ss, 1e-30)
            yi = 0x5F3759DF - (lax.bitcast_convert_type(sc, jnp.int32) >> 1)
            y = lax.bitcast_convert_type(yi, jnp.float32)
            hs = 0.5 * sc
            y = y * (1.5 - hs * y * y)
            y = y * (1.5 - hs * y * y)
            eu = ss * y
            em = eu - 1.0
            ev = em * em
            gi = ((vbs & _LO) << 4) + lane
            plsc.addupdate_scatter(bins_ev, [gi], ev)
            plsc.addupdate_scatter(bins_cnt, [gi], ones)
            return carry2

        lax.fori_loop(0, _CHUNK // _L, _vec, 0)

    _start(_idx_copies(0, sidx0, didx0, semB0))
    _wait(_idx_copies(0, sidx0, didx0, semB0))
    _start(_gather_copies(sidx0, didx0, cols0, semA0))
    _start(_idx_copies(1, sidx1, didx1, semB1))

    def _pipe(i, carry):
        n0 = 2 * i
        n1 = n0 + 1

        @pl.when(_valid(n1))
        def _s1():
            _wait(_idx_copies(n1, sidx1, didx1, semB1))
            _start(_gather_copies(sidx1, didx1, cols1, semA1))

        @pl.when(_valid(n0))
        def _s2():
            _wait(_gather_copies(sidx0, didx0, cols0, semA0))

        @pl.when(_valid(n0 + 2))
        def _s3():
            _start(_idx_copies(n0 + 2, sidx0, didx0, semB0))

        @pl.when(_valid(n0))
        def _s4():
            _compute(cols0)

        @pl.when(_valid(n0 + 2))
        def _s5():
            _wait(_idx_copies(n0 + 2, sidx0, didx0, semB0))
            _start(_gather_copies(sidx0, didx0, cols0, semA0))

        @pl.when(_valid(n1))
        def _s6():
            _wait(_gather_copies(sidx1, didx1, cols1, semA1))

        @pl.when(_valid(n1 + 2))
        def _s7():
            _start(_idx_copies(n1 + 2, sidx1, didx1, semB1))

        @pl.when(_valid(n1))
        def _s8():
            _compute(cols1)

        return carry

    lax.fori_loop(0, _CPW // 2, _pipe, 0)

    pltpu.sync_copy(bins_ev, out_hbm.at[w, 0])
    pltpu.sync_copy(bins_cnt, out_hbm.at[w, 1])


def _finish_body(ev_ref, cnt_ref, o_ref):
    ev = jnp.sum(ev_ref[...], axis=0)
    cnt = jnp.sum(cnt_ref[...], axis=0)
    evg = jnp.sum(ev, axis=1, keepdims=True)
    cg = jnp.sum(cnt, axis=1, keepdims=True)
    gv = jnp.where(cg > 0, evg / jnp.maximum(cg, 1.0), 0.0)
    o_ref[...] = (jnp.sum(gv) / jnp.float32(_N_GRAPHS)).reshape(1, 1)


@jax.jit
def _run(ta, tb, src1d, dst1d):
    mesh = plsc.VectorSubcoreMesh(core_axis_name="c", subcore_axis_name="s")
    partials = pl.kernel(
        _sc_body,
        out_type=jax.ShapeDtypeStruct((_NW, 2, _N_GRAPHS * _L), jnp.float32),
        mesh=mesh,
        compiler_params=pltpu.CompilerParams(needs_layout_passes=False),
        scratch_types=[
            pltpu.VMEM_SHARED((_N_NODES,), jnp.int32),
            pltpu.VMEM_SHARED((_N_NODES,), jnp.int32),
            pltpu.VMEM((_CHUNK,), jnp.int32),
            pltpu.VMEM((_CHUNK,), jnp.int32),
            pltpu.VMEM((_CHUNK,), jnp.int32),
            pltpu.VMEM((_CHUNK,), jnp.int32),
            pltpu.VMEM((4 * _CHUNK,), jnp.int32),
            pltpu.VMEM((4 * _CHUNK,), jnp.int32),
            pltpu.VMEM((_N_GRAPHS * _L,), jnp.float32),
            pltpu.VMEM((_N_GRAPHS * _L,), jnp.float32),
            pltpu.SemaphoreType.DMA,
            pltpu.SemaphoreType.DMA,
            pltpu.SemaphoreType.DMA,
            pltpu.SemaphoreType.DMA,
        ],
    )(ta, tb, src1d, dst1d)

    ev_part = partials[:, 0, :].reshape(_NW, _N_GRAPHS, _L)
    cnt_part = partials[:, 1, :].reshape(_NW, _N_GRAPHS, _L)
    res = pl.pallas_call(
        _finish_body,
        out_shape=jax.ShapeDtypeStruct((1, 1), jnp.float32),
    )(ev_part, cnt_part)
    return res[0, 0]


def kernel(node_pos, raw_edge_index, batch):
    ei = raw_edge_index.astype(jnp.int32)
    pos = node_pos.astype(jnp.float32)
    xb = lax.bitcast_convert_type(pos[:, 0].astype(jnp.bfloat16),
                                  jnp.uint16).astype(jnp.uint32)
    yb = lax.bitcast_convert_type(pos[:, 1].astype(jnp.bfloat16),
                                  jnp.uint16).astype(jnp.uint32)
    zb = lax.bitcast_convert_type(pos[:, 2].astype(jnp.bfloat16),
                                  jnp.uint16).astype(jnp.uint32)
    ta = lax.bitcast_convert_type((yb << 16) | xb, jnp.int32)
    tb = lax.bitcast_convert_type((zb << 16) | batch.astype(jnp.uint32),
                                  jnp.int32)
    return _run(ta, tb, ei[0], ei[1])


# speedup vs baseline: 2.7584x; 1.0009x over previous
import functools

import jax
import jax.numpy as jnp
from jax import lax
from jax.experimental import pallas as pl
from jax.experimental.pallas import tpu as pltpu
from jax.experimental.pallas import tpu_sc as plsc

_N_NODES = 100000
_N_EDGES = 6400000
_N_GRAPHS = 128

_NC, _NS, _L = 2, 16, 16
_NW = _NC * _NS
_CHUNK = 2048
_NCHUNKS = _N_EDGES // _CHUNK
_CPW = -(-_NCHUNKS // _NW)


def _sc_body(ta_h, tb_h, src_hbm, dst_hbm, out_hbm,
             ta, tb, sidx0, didx0, sidx1, didx1,
             cols0, cols1, bins_ev, bins_cnt,
             semA0, semA1, semB0, semB1):
    c = lax.axis_index("c")
    s = lax.axis_index("s")
    w = s * _NC + c

    @pl.when(s == 0)
    def _stage():
        pltpu.sync_copy(ta_h, ta)
        pltpu.sync_copy(tb_h, tb)

    plsc.subcore_barrier()

    zeros = jnp.zeros((_L,), jnp.float32)

    def _zero(i, carry):
        bins_ev[pl.ds(i * _L, _L)] = zeros
        bins_cnt[pl.ds(i * _L, _L)] = zeros
        return carry

    lax.fori_loop(0, _N_GRAPHS, _zero, 0)

    lane = lax.iota(jnp.int32, _L)
    ones = jnp.ones((_L,), jnp.float32)

    def _valid(n):
        return (w + n * _NW) < _NCHUNKS

    def _idx_copies(n, sidx_, didx_, semb):
        base = (w + n * _NW) * _CHUNK
        return [
            pltpu.make_async_copy(src_hbm.at[pl.ds(base, _CHUNK)], sidx_,
                                  semb),
            pltpu.make_async_copy(dst_hbm.at[pl.ds(base, _CHUNK)], didx_,
                                  semb),
        ]

    def _gather_copies(sidx_, didx_, cols, sema):
        return [
            pltpu.make_async_copy(ta.at[sidx_], cols.at[pl.ds(0, _CHUNK)],
                                  sema),
            pltpu.make_async_copy(tb.at[sidx_],
                                  cols.at[pl.ds(_CHUNK, _CHUNK)], sema),
            pltpu.make_async_copy(ta.at[didx_],
                                  cols.at[pl.ds(2 * _CHUNK, _CHUNK)], sema),
            pltpu.make_async_copy(tb.at[didx_],
                                  cols.at[pl.ds(3 * _CHUNK, _CHUNK)], sema),
        ]

    def _start(copies):
        for cp in copies:
            cp.start()

    def _wait(copies):
        for cp in copies:
            cp.wait()

    _HI = jnp.int32(-65536)
    _LO = jnp.int32(65535)

    def _bcf(v):
        return lax.bitcast_convert_type(v, jnp.float32)

    def _compute(cols):
        def _vec(k, carry2):
            vas = cols[pl.ds(k * _L, _L)]
            vbs = cols[pl.ds(_CHUNK + k * _L, _L)]
            vad = cols[pl.ds(2 * _CHUNK + k * _L, _L)]
            vbd = cols[pl.ds(3 * _CHUNK + k * _L, _L)]
            dx = _bcf(vad << 16) - _bcf(vas << 16)
            dy = _bcf(vad & _HI) - _bcf(vas & _HI)
            dz = _bcf(vbd & _HI) - _bcf(vbs & _HI)
            ss = dx * dx + dy * dy + dz * dz
            sc = jnp.maximum(ss, 1e-30)
            yi = 0x5F3759DF - (lax.bitcast_convert_type(sc, jnp.int32) >> 1)
            y = lax.bitcast_convert_type(yi, jnp.float32)
            hs = 0.5 * sc
            y = y * (1.5 - hs * y * y)
            y = y * (1.5 - hs * y * y)
            eu = ss * y
            em = eu - 1.0
            ev = em * em
            gi = ((vbs & _LO) << 4) + lane
            plsc.addupdate_scatter(bins_ev, [gi], ev)
            plsc.addupdate_scatter(bins_cnt, [gi], ones)
            return carry2

        lax.fori_loop(0, _CHUNK // _L, _vec, 0)

    _start(_idx_copies(0, sidx0, didx0, semB0))
    _wait(_idx_copies(0, sidx0, didx0, semB0))
    _start(_gather_copies(sidx0, didx0, cols0, semA0))
    _start(_idx_copies(1, sidx1, didx1, semB1))

    def _pipe(i, carry):
        n0 = 2 * i
        n1 = n0 + 1

        @pl.when(_valid(n1))
        def _s1():
            _wait(_idx_copies(n1, sidx1, didx1, semB1))
            _start(_gather_copies(sidx1, didx1, cols1, semA1))

        @pl.when(_valid(n0))
        def _s2():
            _wait(_gather_copies(sidx0, didx0, cols0, semA0))

        @pl.when(_valid(n0 + 2))
        def _s3():
            _start(_idx_copies(n0 + 2, sidx0, didx0, semB0))

        @pl.when(_valid(n0))
        def _s4():
            _compute(cols0)

        @pl.when(_valid(n0 + 2))
        def _s5():
            _wait(_idx_copies(n0 + 2, sidx0, didx0, semB0))
            _start(_gather_copies(sidx0, didx0, cols0, semA0))

        @pl.when(_valid(n1))
        def _s6():
            _wait(_gather_copies(sidx1, didx1, cols1, semA1))

        @pl.when(_valid(n1 + 2))
        def _s7():
            _start(_idx_copies(n1 + 2, sidx1, didx1, semB1))

        @pl.when(_valid(n1))
        def _s8():
            _compute(cols1)

        return carry

    lax.fori_loop(0, -(-_CPW // 2), _pipe, 0)

    pltpu.sync_copy(bins_ev, out_hbm.at[w, 0])
    pltpu.sync_copy(bins_cnt, out_hbm.at[w, 1])


def _finish_body(ev_ref, cnt_ref, o_ref):
    ev = jnp.sum(ev_ref[...], axis=0)
    cnt = jnp.sum(cnt_ref[...], axis=0)
    evg = jnp.sum(ev, axis=1, keepdims=True)
    cg = jnp.sum(cnt, axis=1, keepdims=True)
    gv = jnp.where(cg > 0, evg / jnp.maximum(cg, 1.0), 0.0)
    o_ref[...] = (jnp.sum(gv) / jnp.float32(_N_GRAPHS)).reshape(1, 1)


@jax.jit
def _run(ta, tb, src1d, dst1d):
    mesh = plsc.VectorSubcoreMesh(core_axis_name="c", subcore_axis_name="s")
    partials = pl.kernel(
        _sc_body,
        out_type=jax.ShapeDtypeStruct((_NW, 2, _N_GRAPHS * _L), jnp.float32),
        mesh=mesh,
        compiler_params=pltpu.CompilerParams(needs_layout_passes=False),
        scratch_types=[
            pltpu.VMEM_SHARED((_N_NODES,), jnp.int32),
            pltpu.VMEM_SHARED((_N_NODES,), jnp.int32),
            pltpu.VMEM((_CHUNK,), jnp.int32),
            pltpu.VMEM((_CHUNK,), jnp.int32),
            pltpu.VMEM((_CHUNK,), jnp.int32),
            pltpu.VMEM((_CHUNK,), jnp.int32),
            pltpu.VMEM((4 * _CHUNK,), jnp.int32),
            pltpu.VMEM((4 * _CHUNK,), jnp.int32),
            pltpu.VMEM((_N_GRAPHS * _L,), jnp.float32),
            pltpu.VMEM((_N_GRAPHS * _L,), jnp.float32),
            pltpu.SemaphoreType.DMA,
            pltpu.SemaphoreType.DMA,
            pltpu.SemaphoreType.DMA,
            pltpu.SemaphoreType.DMA,
        ],
    )(ta, tb, src1d, dst1d)

    ev_part = partials[:, 0, :].reshape(_NW, _N_GRAPHS, _L)
    cnt_part = partials[:, 1, :].reshape(_NW, _N_GRAPHS, _L)
    res = pl.pallas_call(
        _finish_body,
        out_shape=jax.ShapeDtypeStruct((1, 1), jnp.float32),
    )(ev_part, cnt_part)
    return res[0, 0]


def kernel(node_pos, raw_edge_index, batch):
    ei = raw_edge_index.astype(jnp.int32)
    pos = node_pos.astype(jnp.float32)
    xb = lax.bitcast_convert_type(pos[:, 0].astype(jnp.bfloat16),
                                  jnp.uint16).astype(jnp.uint32)
    yb = lax.bitcast_convert_type(pos[:, 1].astype(jnp.bfloat16),
                                  jnp.uint16).astype(jnp.uint32)
    zb = lax.bitcast_convert_type(pos[:, 2].astype(jnp.bfloat16),
                                  jnp.uint16).astype(jnp.uint32)
    ta = lax.bitcast_convert_type((yb << 16) | xb, jnp.int32)
    tb = lax.bitcast_convert_type((zb << 16) | batch.astype(jnp.uint32),
                                  jnp.int32)
    return _run(ta, tb, ei[0], ei[1])


# speedup vs baseline: 2.9154x; 1.0569x over previous
import functools

import jax
import jax.numpy as jnp
from jax import lax
from jax.experimental import pallas as pl
from jax.experimental.pallas import tpu as pltpu
from jax.experimental.pallas import tpu_sc as plsc

_N_NODES = 100000
_N_EDGES = 6400000
_N_GRAPHS = 128

_NC, _NS, _L = 2, 16, 16
_NW = _NC * _NS
_CHUNK = 2048
_NCHUNKS = _N_EDGES // _CHUNK
_CPW = -(-_NCHUNKS // _NW)


def _sc_body(ta_h, tb_h, edges_hbm, out_hbm,
             ta, tb, sidx0, didx0, sidx1, didx1,
             cols0, cols1, bins_ev, bins_cnt,
             semA0, semA1, semB0, semB1):
    c = lax.axis_index("c")
    s = lax.axis_index("s")
    w = s * _NC + c

    @pl.when(s == 0)
    def _stage():
        pltpu.sync_copy(ta_h, ta)
        pltpu.sync_copy(tb_h, tb)

    plsc.subcore_barrier()

    zeros = jnp.zeros((_L,), jnp.float32)

    def _zero(i, carry):
        bins_ev[pl.ds(i * _L, _L)] = zeros
        bins_cnt[pl.ds(i * _L, _L)] = zeros
        return carry

    lax.fori_loop(0, _N_GRAPHS, _zero, 0)

    lane = lax.iota(jnp.int32, _L)
    ones = jnp.ones((_L,), jnp.float32)

    def _valid(n):
        return (w + n * _NW) < _NCHUNKS

    def _idx_copies(n, sidx_, didx_, semb):
        base = (w + n * _NW) * _CHUNK
        return [
            pltpu.make_async_copy(edges_hbm.at[pl.ds(base, _CHUNK)], sidx_,
                                  semb),
            pltpu.make_async_copy(
                edges_hbm.at[pl.ds(_N_EDGES + base, _CHUNK)], didx_, semb),
        ]

    def _gather_copies(sidx_, didx_, cols, sema):
        return [
            pltpu.make_async_copy(ta.at[sidx_], cols.at[pl.ds(0, _CHUNK)],
                                  sema),
            pltpu.make_async_copy(tb.at[sidx_],
                                  cols.at[pl.ds(_CHUNK, _CHUNK)], sema),
            pltpu.make_async_copy(ta.at[didx_],
                                  cols.at[pl.ds(2 * _CHUNK, _CHUNK)], sema),
            pltpu.make_async_copy(tb.at[didx_],
                                  cols.at[pl.ds(3 * _CHUNK, _CHUNK)], sema),
        ]

    def _start(copies):
        for cp in copies:
            cp.start()

    def _wait(copies):
        for cp in copies:
            cp.wait()

    _HI = jnp.int32(-65536)
    _LO = jnp.int32(65535)

    def _bcf(v):
        return lax.bitcast_convert_type(v, jnp.float32)

    def _compute(cols):
        def _vec(k, carry2):
            vas = cols[pl.ds(k * _L, _L)]
            vbs = cols[pl.ds(_CHUNK + k * _L, _L)]
            vad = cols[pl.ds(2 * _CHUNK + k * _L, _L)]
            vbd = cols[pl.ds(3 * _CHUNK + k * _L, _L)]
            dx = _bcf(vad << 16) - _bcf(vas << 16)
            dy = _bcf(vad & _HI) - _bcf(vas & _HI)
            dz = _bcf(vbd & _HI) - _bcf(vbs & _HI)
            ss = dx * dx + dy * dy + dz * dz
            sc = jnp.maximum(ss, 1e-30)
            yi = 0x5F3759DF - (lax.bitcast_convert_type(sc, jnp.int32) >> 1)
            y = lax.bitcast_convert_type(yi, jnp.float32)
            hs = 0.5 * sc
            y = y * (1.5 - hs * y * y)
            y = y * (1.5 - hs * y * y)
            eu = ss * y
            em = eu - 1.0
            ev = em * em
            gi = ((vbs & _LO) << 4) + lane
            plsc.addupdate_scatter(bins_ev, [gi], ev)
            plsc.addupdate_scatter(bins_cnt, [gi], ones)
            return carry2

        lax.fori_loop(0, _CHUNK // _L, _vec, 0)

    _start(_idx_copies(0, sidx0, didx0, semB0))
    _wait(_idx_copies(0, sidx0, didx0, semB0))
    _start(_gather_copies(sidx0, didx0, cols0, semA0))
    _start(_idx_copies(1, sidx1, didx1, semB1))

    def _pipe(i, carry):
        n0 = 2 * i
        n1 = n0 + 1

        @pl.when(_valid(n1))
        def _s1():
            _wait(_idx_copies(n1, sidx1, didx1, semB1))
            _start(_gather_copies(sidx1, didx1, cols1, semA1))

        @pl.when(_valid(n0))
        def _s2():
            _wait(_gather_copies(sidx0, didx0, cols0, semA0))

        @pl.when(_valid(n0 + 2))
        def _s3():
            _start(_idx_copies(n0 + 2, sidx0, didx0, semB0))

        @pl.when(_valid(n0))
        def _s4():
            _compute(cols0)

        @pl.when(_valid(n0 + 2))
        def _s5():
            _wait(_idx_copies(n0 + 2, sidx0, didx0, semB0))
            _start(_gather_copies(sidx0, didx0, cols0, semA0))

        @pl.when(_valid(n1))
        def _s6():
            _wait(_gather_copies(sidx1, didx1, cols1, semA1))

        @pl.when(_valid(n1 + 2))
        def _s7():
            _start(_idx_copies(n1 + 2, sidx1, didx1, semB1))

        @pl.when(_valid(n1))
        def _s8():
            _compute(cols1)

        return carry

    lax.fori_loop(0, -(-_CPW // 2), _pipe, 0)

    pltpu.sync_copy(bins_ev, out_hbm.at[w, 0])
    pltpu.sync_copy(bins_cnt, out_hbm.at[w, 1])


def _finish_body(ev_ref, cnt_ref, o_ref):
    ev = jnp.sum(ev_ref[...], axis=0)
    cnt = jnp.sum(cnt_ref[...], axis=0)
    evg = jnp.sum(ev, axis=1, keepdims=True)
    cg = jnp.sum(cnt, axis=1, keepdims=True)
    gv = jnp.where(cg > 0, evg / jnp.maximum(cg, 1.0), 0.0)
    o_ref[...] = (jnp.sum(gv) / jnp.float32(_N_GRAPHS)).reshape(1, 1)


@jax.jit
def _run(ta, tb, edges1d):
    mesh = plsc.VectorSubcoreMesh(core_axis_name="c", subcore_axis_name="s")
    partials = pl.kernel(
        _sc_body,
        out_type=jax.ShapeDtypeStruct((_NW, 2, _N_GRAPHS * _L), jnp.float32),
        mesh=mesh,
        compiler_params=pltpu.CompilerParams(needs_layout_passes=False),
        scratch_types=[
            pltpu.VMEM_SHARED((_N_NODES,), jnp.int32),
            pltpu.VMEM_SHARED((_N_NODES,), jnp.int32),
            pltpu.VMEM((_CHUNK,), jnp.int32),
            pltpu.VMEM((_CHUNK,), jnp.int32),
            pltpu.VMEM((_CHUNK,), jnp.int32),
            pltpu.VMEM((_CHUNK,), jnp.int32),
            pltpu.VMEM((4 * _CHUNK,), jnp.int32),
            pltpu.VMEM((4 * _CHUNK,), jnp.int32),
            pltpu.VMEM((_N_GRAPHS * _L,), jnp.float32),
            pltpu.VMEM((_N_GRAPHS * _L,), jnp.float32),
            pltpu.SemaphoreType.DMA,
            pltpu.SemaphoreType.DMA,
            pltpu.SemaphoreType.DMA,
            pltpu.SemaphoreType.DMA,
        ],
    )(ta, tb, edges1d)

    ev_part = partials[:, 0, :].reshape(_NW, _N_GRAPHS, _L)
    cnt_part = partials[:, 1, :].reshape(_NW, _N_GRAPHS, _L)
    res = pl.pallas_call(
        _finish_body,
        out_shape=jax.ShapeDtypeStruct((1, 1), jnp.float32),
    )(ev_part, cnt_part)
    return res[0, 0]


def kernel(node_pos, raw_edge_index, batch):
    ei = raw_edge_index.astype(jnp.int32)
    pos = node_pos.astype(jnp.float32)
    xb = lax.bitcast_convert_type(pos[:, 0].astype(jnp.bfloat16),
                                  jnp.uint16).astype(jnp.uint32)
    yb = lax.bitcast_convert_type(pos[:, 1].astype(jnp.bfloat16),
                                  jnp.uint16).astype(jnp.uint32)
    zb = lax.bitcast_convert_type(pos[:, 2].astype(jnp.bfloat16),
                                  jnp.uint16).astype(jnp.uint32)
    ta = lax.bitcast_convert_type((yb << 16) | xb, jnp.int32)
    tb = lax.bitcast_convert_type((zb << 16) | batch.astype(jnp.uint32),
                                  jnp.int32)
    return _run(ta, tb, ei.reshape(2 * _N_EDGES))


# speedup vs baseline: 2.9175x; 1.0007x over previous
import jax
import jax.numpy as jnp
from jax import lax
from jax.experimental import pallas as pl
from jax.experimental.pallas import tpu as pltpu
from jax.experimental.pallas import tpu_sc as plsc

_N_NODES = 100000
_N_EDGES = 6400000
_N_GRAPHS = 128

_NC, _NS, _L = 2, 16, 16
_NW = _NC * _NS
_CHUNK = 2048
_NCHUNKS = _N_EDGES // _CHUNK
_CPW = -(-_NCHUNKS // _NW)


def _sc_body(ta_h, tb_h, edges_hbm, out_hbm,
             ta, tb, sidx0, didx0, sidx1, didx1,
             cols0, cols1, bins_ev, bins_cnt,
             semA0, semA1, semB0, semB1):
    c = lax.axis_index("c")
    s = lax.axis_index("s")
    w = s * _NC + c

    @pl.when(s == 0)
    def _stage():
        pltpu.sync_copy(ta_h, ta)
        pltpu.sync_copy(tb_h, tb)

    plsc.subcore_barrier()

    zeros = jnp.zeros((_L,), jnp.float32)

    def _zero(i, carry):
        bins_ev[pl.ds(i * _L, _L)] = zeros
        bins_cnt[pl.ds(i * _L, _L)] = zeros
        return carry

    lax.fori_loop(0, _N_GRAPHS, _zero, 0)

    lane = lax.iota(jnp.int32, _L)
    ones = jnp.ones((_L,), jnp.float32)

    def _valid(n):
        return (w + n * _NW) < _NCHUNKS

    def _idx_copies(n, sidx_, didx_, semb):
        base = (w + n * _NW) * _CHUNK
        return [
            pltpu.make_async_copy(edges_hbm.at[pl.ds(base, _CHUNK)], sidx_,
                                  semb),
            pltpu.make_async_copy(
                edges_hbm.at[pl.ds(_N_EDGES + base, _CHUNK)], didx_, semb),
        ]

    def _gather_copies(sidx_, didx_, cols, sema):
        return [
            pltpu.make_async_copy(ta.at[sidx_], cols.at[pl.ds(0, _CHUNK)],
                                  sema),
            pltpu.make_async_copy(tb.at[sidx_],
                                  cols.at[pl.ds(_CHUNK, _CHUNK)], sema),
            pltpu.make_async_copy(ta.at[didx_],
                                  cols.at[pl.ds(2 * _CHUNK, _CHUNK)], sema),
            pltpu.make_async_copy(tb.at[didx_],
                                  cols.at[pl.ds(3 * _CHUNK, _CHUNK)], sema),
        ]

    def _start(copies):
        for cp in copies:
            cp.start()

    def _wait(copies):
        for cp in copies:
            cp.wait()

    _HI = jnp.int32(-65536)
    _LO = jnp.int32(65535)

    def _bcf(v):
        return lax.bitcast_convert_type(v, jnp.float32)

    def _compute(cols):
        def _vec(k, carry2):
            vas = cols[pl.ds(k * _L, _L)]
            vbs = cols[pl.ds(_CHUNK + k * _L, _L)]
            vad = cols[pl.ds(2 * _CHUNK + k * _L, _L)]
            vbd = cols[pl.ds(3 * _CHUNK + k * _L, _L)]
            dx = _bcf(vad << 16) - _bcf(vas << 16)
            dy = _bcf(vad & _HI) - _bcf(vas & _HI)
            dz = _bcf(vbd & _HI) - _bcf(vbs & _HI)
            ss = dx * dx + dy * dy + dz * dz
            sc = jnp.maximum(ss, 1e-30)
            yi = 0x5F3759DF - (lax.bitcast_convert_type(sc, jnp.int32) >> 1)
            y = lax.bitcast_convert_type(yi, jnp.float32)
            hs = 0.5 * sc
            y = y * (1.5 - hs * y * y)
            y = y * (1.5 - hs * y * y)
            eu = ss * y
            em = eu - 1.0
            ev = em * em
            gi = ((vbs & _LO) << 4) + lane
            plsc.addupdate_scatter(bins_ev, [gi], ev)
            plsc.addupdate_scatter(bins_cnt, [gi], ones)
            return carry2

        lax.fori_loop(0, _CHUNK // _L, _vec, 0)

    _start(_idx_copies(0, sidx0, didx0, semB0))
    _wait(_idx_copies(0, sidx0, didx0, semB0))
    _start(_gather_copies(sidx0, didx0, cols0, semA0))
    _start(_idx_copies(1, sidx1, didx1, semB1))

    def _pipe(i, carry):
        n0 = 2 * i
        n1 = n0 + 1

        @pl.when(_valid(n1))
        def _s1():
            _wait(_idx_copies(n1, sidx1, didx1, semB1))
            _start(_gather_copies(sidx1, didx1, cols1, semA1))

        @pl.when(_valid(n0))
        def _s2():
            _wait(_gather_copies(sidx0, didx0, cols0, semA0))

        @pl.when(_valid(n0 + 2))
        def _s3():
            _start(_idx_copies(n0 + 2, sidx0, didx0, semB0))

        @pl.when(_valid(n0))
        def _s4():
            _compute(cols0)

        @pl.when(_valid(n0 + 2))
        def _s5():
            _wait(_idx_copies(n0 + 2, sidx0, didx0, semB0))
            _start(_gather_copies(sidx0, didx0, cols0, semA0))

        @pl.when(_valid(n1))
        def _s6():
            _wait(_gather_copies(sidx1, didx1, cols1, semA1))

        @pl.when(_valid(n1 + 2))
        def _s7():
            _start(_idx_copies(n1 + 2, sidx1, didx1, semB1))

        @pl.when(_valid(n1))
        def _s8():
            _compute(cols1)

        return carry

    lax.fori_loop(0, -(-_CPW // 2), _pipe, 0)

    pltpu.sync_copy(bins_ev, out_hbm.at[w, 0])
    pltpu.sync_copy(bins_cnt, out_hbm.at[w, 1])


def _finish_body(ev_ref, cnt_ref, o_ref):
    ev = jnp.sum(ev_ref[...], axis=0)
    cnt = jnp.sum(cnt_ref[...], axis=0)
    evg = jnp.sum(ev, axis=1, keepdims=True)
    cg = jnp.sum(cnt, axis=1, keepdims=True)
    gv = jnp.where(cg > 0, evg / jnp.maximum(cg, 1.0), 0.0)
    o_ref[...] = (jnp.sum(gv) / jnp.float32(_N_GRAPHS)).reshape(1, 1)


@jax.jit
def _run(ta, tb, edges1d):
    mesh = plsc.VectorSubcoreMesh(core_axis_name="c", subcore_axis_name="s")
    partials = pl.kernel(
        _sc_body,
        out_type=jax.ShapeDtypeStruct((_NW, 2, _N_GRAPHS * _L), jnp.float32),
        mesh=mesh,
        compiler_params=pltpu.CompilerParams(needs_layout_passes=False),
        scratch_types=[
            pltpu.VMEM_SHARED((_N_NODES,), jnp.int32),
            pltpu.VMEM_SHARED((_N_NODES,), jnp.int32),
            pltpu.VMEM((_CHUNK,), jnp.int32),
            pltpu.VMEM((_CHUNK,), jnp.int32),
            pltpu.VMEM((_CHUNK,), jnp.int32),
            pltpu.VMEM((_CHUNK,), jnp.int32),
            pltpu.VMEM((4 * _CHUNK,), jnp.int32),
            pltpu.VMEM((4 * _CHUNK,), jnp.int32),
            pltpu.VMEM((_N_GRAPHS * _L,), jnp.float32),
            pltpu.VMEM((_N_GRAPHS * _L,), jnp.float32),
            pltpu.SemaphoreType.DMA,
            pltpu.SemaphoreType.DMA,
            pltpu.SemaphoreType.DMA,
            pltpu.SemaphoreType.DMA,
        ],
    )(ta, tb, edges1d)

    ev_part = partials[:, 0, :].reshape(_NW, _N_GRAPHS, _L)
    cnt_part = partials[:, 1, :].reshape(_NW, _N_GRAPHS, _L)
    res = pl.pallas_call(
        _finish_body,
        out_shape=jax.ShapeDtypeStruct((1, 1), jnp.float32),
    )(ev_part, cnt_part)
    return res[0, 0]


def kernel(node_pos, raw_edge_index, batch):
    ei = raw_edge_index.astype(jnp.int32)
    pos = node_pos.astype(jnp.float32)
    xb = lax.bitcast_convert_type(pos[:, 0].astype(jnp.bfloat16),
                                  jnp.uint16).astype(jnp.uint32)
    yb = lax.bitcast_convert_type(pos[:, 1].astype(jnp.bfloat16),
                                  jnp.uint16).astype(jnp.uint32)
    zb = lax.bitcast_convert_type(pos[:, 2].astype(jnp.bfloat16),
                                  jnp.uint16).astype(jnp.uint32)
    ta = lax.bitcast_convert_type((yb << 16) | xb, jnp.int32)
    tb = lax.bitcast_convert_type((zb << 16) | batch.astype(jnp.uint32),
                                  jnp.int32)
    return _run(ta, tb, ei.reshape(2 * _N_EDGES))
